# baseline (speedup 1.0000x reference)
import functools

import jax
import jax.numpy as jnp
from jax import lax
from jax.experimental import pallas as pl
from jax.experimental.pallas import tpu as pltpu
from jax.experimental.pallas import tpu_sc as plsc

VOCAB_SIZE = 1000000
N_EMBD = 64
BATCH = 4096
SEQ_LEN = 200

NC, NS = 2, 16
NW = NC * NS
ROWS_PW = BATCH // NW
IPW = ROWS_PW * SEQ_LEN
PIDX_W = 256
NG = ROWS_PW
TAIL = 184

_mesh = plsc.VectorSubcoreMesh(
    core_axis_name="c", subcore_axis_name="s", num_cores=NC, num_subcores=NS)


@functools.partial(
    pl.kernel,
    out_type=jax.ShapeDtypeStruct((BATCH, SEQ_LEN, N_EMBD), jnp.float32),
    mesh=_mesh,
    scratch_types=[
        pltpu.VMEM((IPW,), jnp.int32),
        pltpu.VMEM((2 * PIDX_W,), jnp.int32),
        pltpu.VMEM((2, PIDX_W, 2 * N_EMBD), jnp.float32),
        pltpu.VMEM((SEQ_LEN, N_EMBD), jnp.float32),
        pltpu.SemaphoreType.DMA,
        pltpu.SemaphoreType.DMA,
        pltpu.SemaphoreType.DMA,
    ],
)
def _embed_sc(idx_hbm, wv_hbm, out_hbm, idx_v, pidx_v, gb, sb, g0, g1, s0):
    gsems = (g0, g1)
    wid = lax.axis_index("s") * NC + lax.axis_index("c")
    wbase = wid * ROWS_PW
    pltpu.sync_copy(idx_hbm.at[pl.ds(wbase * SEQ_LEN, IPW)], idx_v)

    zeros16 = jnp.zeros((16,), jnp.int32)

    def transform(g, b):
        base = g * SEQ_LEN
        for off in (192, 208, 224):
            pidx_v[pl.ds(b * PIDX_W + off, 16)] = zeros16
        for k in range(12):
            raw = idx_v[pl.ds(base + 16 * k, 16)]
            pidx_v[pl.ds(b * PIDX_W + 16 * k, 16)] = lax.shift_right_logical(raw, 1)
        raw = idx_v[pl.ds(base + TAIL, 16)]
        pidx_v[pl.ds(b * PIDX_W + 240, 16)] = lax.shift_right_logical(raw, 1)

    def fire_g(b):
        for h in range(2):
            pltpu.async_copy(
                wv_hbm.at[pidx_v.at[pl.ds(b * PIDX_W + h * 128, 128)]],
                gb.at[b, pl.ds(h * 128, 128)], gsems[b])

    def wait_g(b):
        pltpu.make_async_copy(wv_hbm.at[pl.ds(0, PIDX_W)], gb.at[b],
                              gsems[b]).wait()

    def select_block(g, b, c0, rc0):
        ids = idx_v[pl.ds(g * SEQ_LEN + c0, 16)]
        offs = (ids & 1) * N_EMBD
        for i in range(16):
            off = offs[i]
            for j in range(4):
                sb[c0 + i, pl.ds(16 * j, 16)] = gb[b, rc0 + i,
                                                   pl.ds(off + 16 * j, 16)]

    def select(g, b):
        def blk(k, carry):
            select_block(g, b, 16 * k, 16 * k)
            return carry
        lax.fori_loop(0, 12, blk, 0)
        select_block(g, b, TAIL, 240)

    def fire_s(g):
        pltpu.async_copy(sb, out_hbm.at[wbase + g], s0)

    def wait_s(g):
        pltpu.make_async_copy(sb, out_hbm.at[wbase + g], s0).wait()

    transform(0, 0); fire_g(0)
    transform(1, 1); fire_g(1)
    wait_g(0); select(0, 0); fire_s(0)
    transform(2, 0); fire_g(0)

    def body(p, carry):
        for dg, b in ((1, 1), (2, 0)):
            g = 2 * p + dg
            wait_g(b)
            wait_s(g - 1)
            select(g, b)
            fire_s(g)
            transform(g + 2, b)
            fire_g(b)
        return carry

    lax.fori_loop(0, (NG - 4) // 2, body, 0)

    wait_g(1); wait_s(NG - 4); select(NG - 3, 1); fire_s(NG - 3)
    transform(NG - 1, 1); fire_g(1)
    wait_g(0); wait_s(NG - 3); select(NG - 2, 0); fire_s(NG - 2)
    wait_g(1); wait_s(NG - 2); select(NG - 1, 1); fire_s(NG - 1)
    wait_s(NG - 1)


def kernel(input_ids, weight):
    wv = weight.reshape(VOCAB_SIZE // 2, 2 * N_EMBD)
    return _embed_sc(input_ids.reshape(-1), wv)


# speedup vs baseline: 8.0228x; 8.0228x over previous
import functools

import jax
import jax.numpy as jnp
from jax import lax
from jax.experimental import pallas as pl
from jax.experimental.pallas import tpu as pltpu
from jax.experimental.pallas import tpu_sc as plsc

VOCAB_SIZE = 1000000
N_EMBD = 64
BATCH = 4096
SEQ_LEN = 200

NC, NS = 2, 16
NW = NC * NS
ROWS_PW = BATCH // NW
IPW = ROWS_PW * SEQ_LEN
PAD = 64
NG = ROWS_PW

_mesh = plsc.VectorSubcoreMesh(
    core_axis_name="c", subcore_axis_name="s", num_cores=NC, num_subcores=NS)


@functools.partial(
    pl.kernel,
    out_type=jax.ShapeDtypeStruct((BATCH, SEQ_LEN, N_EMBD), jnp.float32),
    mesh=_mesh,
    scratch_types=[
        pltpu.VMEM((IPW + PAD,), jnp.int32),
        pltpu.VMEM((2, 2 * 128, 2 * N_EMBD), jnp.float32),
        pltpu.VMEM((SEQ_LEN, N_EMBD), jnp.float32),
        pltpu.SemaphoreType.DMA,
        pltpu.SemaphoreType.DMA,
        pltpu.SemaphoreType.DMA,
    ],
)
def _embed_sc(idx_hbm, wv_hbm, out_hbm, idx_v, gb, sb, g0, g1, s0):
    gsems = (g0, g1)
    wid = lax.axis_index("s") * NC + lax.axis_index("c")
    wbase = wid * ROWS_PW
    pltpu.sync_copy(idx_hbm.at[pl.ds(wbase * SEQ_LEN, IPW + PAD)], idx_v)

    def fire_g(g, b):
        for h in range(2):
            pltpu.async_copy(
                wv_hbm.at[idx_v.at[pl.ds(g * SEQ_LEN + h * 128, 128)]],
                gb.at[b, pl.ds(h * 128, 128)], gsems[b])

    def wait_g(b):
        pltpu.make_async_copy(wv_hbm.at[pl.ds(0, 256)], gb.at[b],
                              gsems[b]).wait()

    def vcopy(b):
        def blk(k, carry):
            for i in range(8):
                c = 8 * k + i
                for j in range(4):
                    sb[c, pl.ds(16 * j, 16)] = gb[b, c, pl.ds(16 * j, 16)]
            return carry
        lax.fori_loop(0, SEQ_LEN // 8, blk, 0)

    def fire_s(g):
        pltpu.async_copy(sb, out_hbm.at[wbase + g], s0)

    def wait_s(g):
        pltpu.make_async_copy(sb, out_hbm.at[wbase + g], s0).wait()

    fire_g(0, 0)
    fire_g(1, 1)
    wait_g(0); vcopy(0); fire_g(2, 0); fire_s(0)

    def body(p, carry):
        for dg, b in ((1, 1), (2, 0)):
            g = 2 * p + dg
            wait_g(b)
            wait_s(g - 1)
            vcopy(b)
            fire_g(g + 2, b)
            fire_s(g)
        return carry

    lax.fori_loop(0, (NG - 4) // 2, body, 0)

    wait_g(1); wait_s(NG - 4); vcopy(1); fire_g(NG - 1, 1); fire_s(NG - 3)
    wait_g(0); wait_s(NG - 3); vcopy(0); fire_s(NG - 2)
    wait_g(1); wait_s(NG - 2); vcopy(1); fire_s(NG - 1)
    wait_s(NG - 1)


def kernel(input_ids, weight):
    wv = jnp.pad(weight, ((0, 0), (0, N_EMBD)))
    idx = jnp.pad(input_ids.reshape(-1), (0, PAD))
    return _embed_sc(idx, wv)


# speedup vs baseline: 8.2629x; 1.0299x over previous
import functools

import jax
import jax.numpy as jnp
from jax import lax
from jax.experimental import pallas as pl
from jax.experimental.pallas import tpu as pltpu
from jax.experimental.pallas import tpu_sc as plsc

VOCAB_SIZE = 1000000
N_EMBD = 64
BATCH = 4096
SEQ_LEN = 200

NC, NS = 2, 16
NW = NC * NS
ROWS_PW = BATCH // NW
IPW = ROWS_PW * SEQ_LEN
NG = ROWS_PW
LISTS = (128, 72)

_mesh = plsc.VectorSubcoreMesh(
    core_axis_name="c", subcore_axis_name="s", num_cores=NC, num_subcores=NS)


@functools.partial(
    pl.kernel,
    out_type=jax.ShapeDtypeStruct((BATCH, SEQ_LEN, N_EMBD), jnp.float32),
    mesh=_mesh,
    scratch_types=[
        pltpu.VMEM((IPW,), jnp.int32),
        pltpu.VMEM((3, SEQ_LEN, 2 * N_EMBD), jnp.float32),
        pltpu.VMEM((SEQ_LEN, N_EMBD), jnp.float32),
        pltpu.SemaphoreType.DMA,
        pltpu.SemaphoreType.DMA,
        pltpu.SemaphoreType.DMA,
        pltpu.SemaphoreType.DMA,
    ],
)
def _embed_sc(idx_hbm, wv_hbm, out_hbm, idx_v, gb, sb, g0, g1, g2, s0):
    gsems = (g0, g1, g2)
    wid = lax.axis_index("s") * NC + lax.axis_index("c")
    wbase = wid * ROWS_PW
    pltpu.sync_copy(idx_hbm.at[pl.ds(wbase * SEQ_LEN, IPW)], idx_v)

    def fire_g(g, b):
        off = 0
        for n in LISTS:
            pltpu.async_copy(
                wv_hbm.at[idx_v.at[pl.ds(g * SEQ_LEN + off, n)]],
                gb.at[b, pl.ds(off, n)], gsems[b])
            off += n

    def wait_g(b):
        pltpu.make_async_copy(wv_hbm.at[pl.ds(0, SEQ_LEN)], gb.at[b],
                              gsems[b]).wait()

    def vcopy(b):
        def blk(k, carry):
            for i in range(8):
                c = 8 * k + i
                for j in range(4):
                    sb[c, pl.ds(16 * j, 16)] = gb[b, c, pl.ds(16 * j, 16)]
            return carry
        lax.fori_loop(0, SEQ_LEN // 8, blk, 0)

    def fire_s(g):
        pltpu.async_copy(sb, out_hbm.at[wbase + g], s0)

    def wait_s(g):
        pltpu.make_async_copy(sb, out_hbm.at[wbase + g], s0).wait()

    fire_g(0, 0)
    fire_g(1, 1)
    fire_g(2, 2)
    wait_g(0); vcopy(0); fire_g(3, 0); fire_s(0)
    wait_g(1); wait_s(0); vcopy(1); fire_g(4, 1); fire_s(1)

    def body(p, carry):
        for dg in range(3):
            g = 3 * p + 2 + dg
            b = (2 + dg) % 3
            wait_g(b)
            wait_s(g - 1)
            vcopy(b)
            fire_g(g + 3, b)
            fire_s(g)
        return carry

    lax.fori_loop(0, (NG - 8) // 3, body, 0)

    for g in (NG - 6, NG - 5, NG - 4):
        b = g % 3
        wait_g(b)
        wait_s(g - 1)
        vcopy(b)
        fire_g(g + 3, b)
        fire_s(g)
    for g in (NG - 3, NG - 2, NG - 1):
        b = g % 3
        wait_g(b)
        wait_s(g - 1)
        vcopy(b)
        fire_s(g)
    wait_s(NG - 1)


def kernel(input_ids, weight):
    wv = jnp.pad(weight, ((0, 0), (0, N_EMBD)))
    return _embed_sc(input_ids.reshape(-1), wv)
